# baseline (speedup 1.0000x reference)
import functools

import jax
import jax.numpy as jnp
from jax import lax
from jax.experimental import pallas as pl
from jax.experimental.pallas import tpu as pltpu
from jax.experimental.pallas import tpu_sc as plsc

_T, _D, _H, _DOUT, _E = 2048, 1024, 2048, 1024, 8
_LN_EPS = 1e-5
_BLK = 128
_P = 2 * _T + _E * _BLK
_NB = _P // _BLK
_NW = 32
_GCH = 32
_CCH = 32
_ROWS_PER_W = _P // _NW
_TOK_PER_W = _T // _NW


_CH = 512
_NCH = (2 * _T) // _CH


def _gate_body(x_ref, wg_ref, bg_ref, w1_ref, w2_ref, sa_ref, sb_ref,
               meta_ref, lbl_ref):
    x = x_ref[...]
    logits = jnp.dot(x, wg_ref[...], preferred_element_type=jnp.float32)
    logits = logits + bg_ref[...]
    m = jnp.max(logits, axis=1, keepdims=True)
    p = jnp.exp(logits - m)
    w = p / jnp.sum(p, axis=1, keepdims=True)
    iota = lax.broadcasted_iota(jnp.int32, w.shape, 1)
    w1 = jnp.max(w, axis=1, keepdims=True)
    i1 = jnp.min(jnp.where(w == w1, iota, _E), axis=1, keepdims=True)
    wm = jnp.where(iota == i1, -jnp.inf, w)
    w2 = jnp.max(wm, axis=1, keepdims=True)
    i2 = jnp.min(jnp.where(wm == w2, iota, _E), axis=1, keepdims=True)
    w1_ref[...] = w1
    w2_ref[...] = w2
    s = jnp.sum(w, axis=0, keepdims=True) * (1.0 / _T)
    lbl_ref[...] = jnp.sum(s * s, axis=1, keepdims=True) * _E

    oh1 = (iota == i1).astype(jnp.float32)
    oh2 = (iota == i2).astype(jnp.float32)
    csums = []
    for c in range(_NCH):
        src = oh1 if c < _NCH // 2 else oh2
        r0 = (c % (_NCH // 2)) * _CH
        csums.append(jnp.sum(src[r0:r0 + _CH], axis=0, keepdims=True))
    cm = jnp.concatenate(csums, axis=0)
    ci = lax.broadcasted_iota(jnp.int32, (_NCH, _NCH), 0)
    cj = lax.broadcasted_iota(jnp.int32, (_NCH, _NCH), 1)
    strict = (cj < ci).astype(jnp.float32)
    chunk_off = jnp.dot(strict, cm, preferred_element_type=jnp.float32)
    counts = jnp.sum(cm, axis=0, keepdims=True)
    cpad = jnp.ceil(counts * (1.0 / _BLK)) * _BLK
    ei = lax.broadcasted_iota(jnp.int32, (_E, _E), 0)
    ej = lax.broadcasted_iota(jnp.int32, (_E, _E), 1)
    incl = (ei <= ej).astype(jnp.float32)
    ends = jnp.dot(cpad, incl, preferred_element_type=jnp.float32)
    off = ends - cpad
    meta_ref[...] = jnp.concatenate(
        [off * (1.0 / _BLK), cpad * (1.0 / _BLK)], axis=1).astype(jnp.int32)
    qi = lax.broadcasted_iota(jnp.int32, (_CH, _CH), 0)
    qj = lax.broadcasted_iota(jnp.int32, (_CH, _CH), 1)
    trilx = (qj < qi).astype(jnp.float32)
    base = off + chunk_off
    for c in range(_NCH):
        src = oh1 if c < _NCH // 2 else oh2
        r0 = (c % (_NCH // 2)) * _CH
        blk = src[r0:r0 + _CH]
        within = jnp.dot(trilx, blk, preferred_element_type=jnp.float32)
        slotm = within + base[c:c + 1]
        slot_c = jnp.sum(slotm * blk, axis=1, keepdims=True)
        tgt = sa_ref if c < _NCH // 2 else sb_ref
        tgt[r0:r0 + _CH, :] = slot_c.astype(jnp.int32)


def _gate(x, Wg, bg):
    return pl.pallas_call(
        _gate_body,
        out_shape=(
            jax.ShapeDtypeStruct((_T, 1), jnp.float32),
            jax.ShapeDtypeStruct((_T, 1), jnp.float32),
            jax.ShapeDtypeStruct((_T, 1), jnp.int32),
            jax.ShapeDtypeStruct((_T, 1), jnp.int32),
            jax.ShapeDtypeStruct((1, 2 * _E), jnp.int32),
            jax.ShapeDtypeStruct((1, 1), jnp.float32),
        ),
    )(x, Wg, bg.reshape(1, _E))


def _sc_dispatch(x, sa, sb):
    mesh = plsc.VectorSubcoreMesh(core_axis_name="c", subcore_axis_name="s")

    @functools.partial(
        pl.kernel,
        mesh=mesh,
        out_type=jax.ShapeDtypeStruct((_P, _D), jnp.float32),
        scratch_types=[
            pltpu.VMEM((_TOK_PER_W,), jnp.int32),
            pltpu.VMEM((_TOK_PER_W,), jnp.int32),
            pltpu.VMEM((_TOK_PER_W, _D), jnp.float32),
            pltpu.SemaphoreType.DMA,
            pltpu.SemaphoreType.DMA,
        ],
    )
    def k(x_hbm, sa_hbm, sb_hbm, gx_hbm, ia_v, ib_v, xb_v, sem_a, sem_b):
        wid = lax.axis_index("s") * 2 + lax.axis_index("c")
        tb = wid * _TOK_PER_W
        pltpu.sync_copy(x_hbm.at[pl.ds(tb, _TOK_PER_W)], xb_v)
        pltpu.sync_copy(sa_hbm.at[pl.ds(tb, _TOK_PER_W)], ia_v)
        pltpu.sync_copy(sb_hbm.at[pl.ds(tb, _TOK_PER_W)], ib_v)
        ca = pltpu.async_copy(xb_v, gx_hbm.at[ia_v], sem_a)
        cb = pltpu.async_copy(xb_v, gx_hbm.at[ib_v], sem_b)
        ca.wait()
        cb.wait()

    return k(x, sa, sb)


def _ffn_body(meta_ref, gx_ref, lng_ref, lnb_ref, w1_ref, b1_ref, w2_ref,
              b2_ref, ws_ref, out_ref,
              w1b0, w1b1, w2b0, w2b1, xb0, xb1, wsb0, wsb1, ob0, ob1,
              wsem, lsem, ssem):
    w1bufs, w2bufs = (w1b0, w1b1), (w2b0, w2b1)
    _WSP = 4
    _DQ = _D // _WSP
    _HQ = _H // _WSP

    def wcopies(e, sl):
        cs = []
        for c in range(_WSP):
            cs.append(pltpu.make_async_copy(
                w1_ref.at[e, pl.ds(c * _DQ, _DQ), :],
                w1bufs[sl].at[0, pl.ds(c * _DQ, _DQ), :],
                wsem.at[sl, c]))
            cs.append(pltpu.make_async_copy(
                w2_ref.at[e, pl.ds(c * _HQ, _HQ), :],
                w2bufs[sl].at[0, pl.ds(c * _HQ, _HQ), :],
                wsem.at[sl, _WSP + c]))
        return cs

    def run_expert(e, w1b, w2b):
        row0 = meta_ref[e] * _BLK
        nb = meta_ref[_E + e]

        def ld_copies(j, xb, wsb, sl):
            r = row0 + j * _BLK
            return (pltpu.make_async_copy(gx_ref.at[pl.ds(r, _BLK), :], xb,
                                          lsem.at[sl, 0]),
                    pltpu.make_async_copy(ws_ref.at[pl.ds(r, _BLK), :], wsb,
                                          lsem.at[sl, 1]))

        def st_copy(j, ob, sl):
            r = row0 + j * _BLK
            return pltpu.make_async_copy(ob, out_ref.at[pl.ds(r, _BLK), :],
                                         ssem.at[sl])

        def start_load(j, xb, wsb, sl):
            @pl.when(j < nb)
            def _():
                ca, cb = ld_copies(j, xb, wsb, sl)
                ca.start()
                cb.start()

        start_load(0, xb0, wsb0, 0)
        start_load(1, xb1, wsb1, 1)

        def process(j, xb, wsb, ob, sl):
            @pl.when(j < nb)
            def _():
                ca, cb = ld_copies(j, xb, wsb, sl)
                ca.wait()
                cb.wait()
                xr = xb[...]
                mu = jnp.mean(xr, axis=1, keepdims=True)
                var = jnp.mean((xr - mu) ** 2, axis=1, keepdims=True)
                xh = (xr - mu) * lax.rsqrt(var + _LN_EPS)
                xn = xh * lng_ref[e] + lnb_ref[e]
                h = jnp.dot(xn, w1b[0],
                            preferred_element_type=jnp.float32) + b1_ref[e]
                h = 0.5 * h * (1.0 + lax.erf(h * (2.0 ** -0.5)))
                o = jnp.dot(h, w2b[0],
                            preferred_element_type=jnp.float32) + b2_ref[e]

                @pl.when(j >= 2)
                def _():
                    st_copy(j - 2, ob, sl).wait()

                ob[...] = o * wsb[...]
                start_load(j + 2, xb, wsb, sl)
                st_copy(j, ob, sl).start()

        def pair(jj, carry):
            j0 = jj * 2
            process(j0, xb0, wsb0, ob0, 0)
            process(j0 + 1, xb1, wsb1, ob1, 1)
            return carry

        lax.fori_loop(0, (nb + 1) // 2, pair, 0)

        @pl.when(nb >= 1)
        def _():
            st_copy(0, ob0, 0).wait()

        @pl.when(nb >= 2)
        def _():
            st_copy(0, ob1, 1).wait()

    for c in wcopies(0, 0):
        c.start()
    for e in range(_E):
        sl = e % 2
        if e + 1 < _E:
            for c in wcopies(e + 1, (e + 1) % 2):
                c.start()
        for c in wcopies(e, sl):
            c.wait()
        run_expert(e, w1bufs[sl], w2bufs[sl])


def _ffn(gx, ln_g, ln_b, W1, b1, W2, b2, w_slot, meta):
    grid_spec = pltpu.PrefetchScalarGridSpec(
        num_scalar_prefetch=1,
        grid=(1,),
        in_specs=[
            pl.BlockSpec(memory_space=pl.ANY),
            pl.BlockSpec((_E, _D), lambda i, m: (0, 0)),
            pl.BlockSpec((_E, _D), lambda i, m: (0, 0)),
            pl.BlockSpec(memory_space=pl.ANY),
            pl.BlockSpec((_E, _H), lambda i, m: (0, 0)),
            pl.BlockSpec(memory_space=pl.ANY),
            pl.BlockSpec((_E, _DOUT), lambda i, m: (0, 0)),
            pl.BlockSpec(memory_space=pl.ANY),
        ],
        out_specs=pl.BlockSpec(memory_space=pl.ANY),
        scratch_shapes=[
            pltpu.VMEM((1, _D, _H), jnp.float32),
            pltpu.VMEM((1, _D, _H), jnp.float32),
            pltpu.VMEM((1, _H, _DOUT), jnp.float32),
            pltpu.VMEM((1, _H, _DOUT), jnp.float32),
            pltpu.VMEM((_BLK, _D), jnp.float32),
            pltpu.VMEM((_BLK, _D), jnp.float32),
            pltpu.VMEM((_BLK, 1), jnp.float32),
            pltpu.VMEM((_BLK, 1), jnp.float32),
            pltpu.VMEM((_BLK, _DOUT), jnp.float32),
            pltpu.VMEM((_BLK, _DOUT), jnp.float32),
            pltpu.SemaphoreType.DMA((2, 8)),
            pltpu.SemaphoreType.DMA((2, 2)),
            pltpu.SemaphoreType.DMA((2,)),
        ],
    )
    return pl.pallas_call(
        _ffn_body,
        grid_spec=grid_spec,
        out_shape=jax.ShapeDtypeStruct((_P, _DOUT), jnp.float32),
    )(meta, gx, ln_g, ln_b, W1, b1, W2, b2, w_slot.reshape(_P, 1))


def _sc_combine(go, s_a, s_b):
    mesh = plsc.VectorSubcoreMesh(core_axis_name="c", subcore_axis_name="s")

    @functools.partial(
        pl.kernel,
        mesh=mesh,
        out_type=jax.ShapeDtypeStruct((_T, _DOUT), jnp.float32),
        scratch_types=[
            pltpu.VMEM((_CCH,), jnp.int32),
            pltpu.VMEM((_CCH,), jnp.int32),
            pltpu.VMEM((_CCH, _DOUT), jnp.float32),
            pltpu.VMEM((_CCH, _DOUT), jnp.float32),
            pltpu.SemaphoreType.DMA,
            pltpu.SemaphoreType.DMA,
        ],
    )
    def k(go_hbm, sa_hbm, sb_hbm, out_hbm, ia_v, ib_v, ba_v, bb_v, sema, semb):
        wid = lax.axis_index("s") * 2 + lax.axis_index("c")
        base = wid * _TOK_PER_W

        def chunk(i, carry):
            b = base + i * _CCH
            pltpu.sync_copy(sa_hbm.at[pl.ds(b, _CCH)], ia_v)
            pltpu.sync_copy(sb_hbm.at[pl.ds(b, _CCH)], ib_v)
            cpa = pltpu.async_copy(go_hbm.at[ia_v], ba_v, sema)
            cpb = pltpu.async_copy(go_hbm.at[ib_v], bb_v, semb)
            cpa.wait()
            cpb.wait()

            def row(r, c):
                for j in range(_DOUT // 16):
                    sl = pl.ds(j * 16, 16)
                    ba_v[r, sl] = ba_v[r, sl] + bb_v[r, sl]
                return c

            lax.fori_loop(0, _CCH, row, 0)
            pltpu.sync_copy(ba_v, out_hbm.at[pl.ds(b, _CCH)])
            return carry

        lax.fori_loop(0, _TOK_PER_W // _CCH, chunk, 0)

    return k(go, s_a, s_b)


def kernel(x, Wg, bg, ln_g, ln_b, W1, b1, W2, b2):
    w1, w2, sa, sb, meta, lbl = _gate(x, Wg, bg)
    s_a, s_b = sa.reshape(_T), sb.reshape(_T)
    slot_cat = jnp.concatenate([s_a, s_b])
    w_cat = jnp.concatenate([w1.reshape(_T), w2.reshape(_T)])
    w_slot = jnp.zeros((_P,), jnp.float32).at[slot_cat].set(w_cat)
    gx = _sc_dispatch(x, s_a, s_b)
    go = _ffn(gx, ln_g, ln_b, W1, b1, W2, b2, w_slot, meta.reshape(2 * _E))
    final = _sc_combine(go, s_a, s_b)
    return final, lbl[0, 0]


# speedup vs baseline: 1.0376x; 1.0376x over previous
import functools

import jax
import jax.numpy as jnp
from jax import lax
from jax.experimental import pallas as pl
from jax.experimental.pallas import tpu as pltpu
from jax.experimental.pallas import tpu_sc as plsc

_T, _D, _H, _DOUT, _E = 2048, 1024, 2048, 1024, 8
_LN_EPS = 1e-5
_BLK = 128
_P = 2 * _T + _E * _BLK
_NB = _P // _BLK
_NW = 32
_GCH = 32
_CCH = 32
_ROWS_PER_W = _P // _NW
_TOK_PER_W = _T // _NW


_CH = 512
_NCH = (2 * _T) // _CH


def _gate_body(x_ref, wg_ref, bg_ref, w1_ref, w2_ref, sa_ref, sb_ref,
               be_ref, lbl_ref):
    x = x_ref[...]
    logits = jnp.dot(x, wg_ref[...], preferred_element_type=jnp.float32)
    logits = logits + bg_ref[...]
    m = jnp.max(logits, axis=1, keepdims=True)
    p = jnp.exp(logits - m)
    w = p / jnp.sum(p, axis=1, keepdims=True)
    iota = lax.broadcasted_iota(jnp.int32, w.shape, 1)
    w1 = jnp.max(w, axis=1, keepdims=True)
    i1 = jnp.min(jnp.where(w == w1, iota, _E), axis=1, keepdims=True)
    wm = jnp.where(iota == i1, -jnp.inf, w)
    w2 = jnp.max(wm, axis=1, keepdims=True)
    i2 = jnp.min(jnp.where(wm == w2, iota, _E), axis=1, keepdims=True)
    w1_ref[...] = w1
    w2_ref[...] = w2
    s = jnp.sum(w, axis=0, keepdims=True) * (1.0 / _T)
    lbl_ref[...] = jnp.sum(s * s, axis=1, keepdims=True) * _E

    oh1 = (iota == i1).astype(jnp.float32)
    oh2 = (iota == i2).astype(jnp.float32)
    csums = []
    for c in range(_NCH):
        src = oh1 if c < _NCH // 2 else oh2
        r0 = (c % (_NCH // 2)) * _CH
        csums.append(jnp.sum(src[r0:r0 + _CH], axis=0, keepdims=True))
    cm = jnp.concatenate(csums, axis=0)
    ci = lax.broadcasted_iota(jnp.int32, (_NCH, _NCH), 0)
    cj = lax.broadcasted_iota(jnp.int32, (_NCH, _NCH), 1)
    strict = (cj < ci).astype(jnp.float32)
    chunk_off = jnp.dot(strict, cm, preferred_element_type=jnp.float32)
    counts = jnp.sum(cm, axis=0, keepdims=True)
    cpad = jnp.ceil(counts * (1.0 / _BLK)) * _BLK
    ei = lax.broadcasted_iota(jnp.int32, (_E, _E), 0)
    ej = lax.broadcasted_iota(jnp.int32, (_E, _E), 1)
    incl = (ei <= ej).astype(jnp.float32)
    ends = jnp.dot(cpad, incl, preferred_element_type=jnp.float32)
    off = ends - cpad
    bstart = lax.broadcasted_iota(jnp.int32, (_NB, _E), 0).astype(jnp.float32) * _BLK
    be = jnp.sum((bstart >= ends).astype(jnp.float32), axis=1, keepdims=True)
    be_ref[...] = jnp.minimum(be, _E - 1.0).astype(jnp.int32)
    qi = lax.broadcasted_iota(jnp.int32, (_CH, _CH), 0)
    qj = lax.broadcasted_iota(jnp.int32, (_CH, _CH), 1)
    trilx = (qj < qi).astype(jnp.float32)
    base = off + chunk_off
    for c in range(_NCH):
        src = oh1 if c < _NCH // 2 else oh2
        r0 = (c % (_NCH // 2)) * _CH
        blk = src[r0:r0 + _CH]
        within = jnp.dot(trilx, blk, preferred_element_type=jnp.float32)
        slotm = within + base[c:c + 1]
        slot_c = jnp.sum(slotm * blk, axis=1, keepdims=True)
        tgt = sa_ref if c < _NCH // 2 else sb_ref
        tgt[r0:r0 + _CH, :] = slot_c.astype(jnp.int32)


def _gate(x, Wg, bg):
    return pl.pallas_call(
        _gate_body,
        out_shape=(
            jax.ShapeDtypeStruct((_T, 1), jnp.float32),
            jax.ShapeDtypeStruct((_T, 1), jnp.float32),
            jax.ShapeDtypeStruct((_T, 1), jnp.int32),
            jax.ShapeDtypeStruct((_T, 1), jnp.int32),
            jax.ShapeDtypeStruct((_NB, 1), jnp.int32),
            jax.ShapeDtypeStruct((1, 1), jnp.float32),
        ),
    )(x, Wg, bg.reshape(1, _E))


def _sc_dispatch(x, sa, sb):
    mesh = plsc.VectorSubcoreMesh(core_axis_name="c", subcore_axis_name="s")

    @functools.partial(
        pl.kernel,
        mesh=mesh,
        out_type=jax.ShapeDtypeStruct((_P, _D), jnp.float32),
        scratch_types=[
            pltpu.VMEM((_TOK_PER_W,), jnp.int32),
            pltpu.VMEM((_TOK_PER_W,), jnp.int32),
            pltpu.VMEM((_TOK_PER_W, _D), jnp.float32),
            pltpu.SemaphoreType.DMA,
            pltpu.SemaphoreType.DMA,
        ],
    )
    def k(x_hbm, sa_hbm, sb_hbm, gx_hbm, ia_v, ib_v, xb_v, sem_a, sem_b):
        wid = lax.axis_index("s") * 2 + lax.axis_index("c")
        tb = wid * _TOK_PER_W
        pltpu.sync_copy(x_hbm.at[pl.ds(tb, _TOK_PER_W)], xb_v)
        pltpu.sync_copy(sa_hbm.at[pl.ds(tb, _TOK_PER_W)], ia_v)
        pltpu.sync_copy(sb_hbm.at[pl.ds(tb, _TOK_PER_W)], ib_v)
        ca = pltpu.async_copy(xb_v, gx_hbm.at[ia_v], sem_a)
        cb = pltpu.async_copy(xb_v, gx_hbm.at[ib_v], sem_b)
        ca.wait()
        cb.wait()

    return k(x, sa, sb)


def _ffn_body(be_ref, gx_ref, lng_ref, lnb_ref, w1_ref, b1_ref, w2_ref,
              b2_ref, ws_ref, out_ref):
    xr = gx_ref[...]
    mu = jnp.mean(xr, axis=1, keepdims=True)
    var = jnp.mean((xr - mu) ** 2, axis=1, keepdims=True)
    xh = (xr - mu) * lax.rsqrt(var + _LN_EPS)
    xn = xh * lng_ref[0, 0] + lnb_ref[0, 0]
    h = jnp.dot(xn, w1_ref[0], preferred_element_type=jnp.float32) + b1_ref[0, 0]
    h = 0.5 * h * (1.0 + lax.erf(h * (2.0 ** -0.5)))
    o = jnp.dot(h, w2_ref[0], preferred_element_type=jnp.float32) + b2_ref[0, 0]
    out_ref[...] = o * ws_ref[...]


def _ffn(gx, ln_g, ln_b, W1, b1, W2, b2, w_slot, block_expert):
    grid_spec = pltpu.PrefetchScalarGridSpec(
        num_scalar_prefetch=1,
        grid=(_NB,),
        in_specs=[
            pl.BlockSpec((_BLK, _D), lambda i, be: (i, 0)),
            pl.BlockSpec((1, 1, _D), lambda i, be: (be[i], 0, 0)),
            pl.BlockSpec((1, 1, _D), lambda i, be: (be[i], 0, 0)),
            pl.BlockSpec((1, _D, _H), lambda i, be: (be[i], 0, 0)),
            pl.BlockSpec((1, 1, _H), lambda i, be: (be[i], 0, 0)),
            pl.BlockSpec((1, _H, _DOUT), lambda i, be: (be[i], 0, 0)),
            pl.BlockSpec((1, 1, _DOUT), lambda i, be: (be[i], 0, 0)),
            pl.BlockSpec((_BLK, 1), lambda i, be: (i, 0)),
        ],
        out_specs=pl.BlockSpec((_BLK, _DOUT), lambda i, be: (i, 0)),
    )
    return pl.pallas_call(
        _ffn_body,
        grid_spec=grid_spec,
        out_shape=jax.ShapeDtypeStruct((_P, _DOUT), jnp.float32),
    )(block_expert, gx, ln_g.reshape(_E, 1, _D), ln_b.reshape(_E, 1, _D),
      W1, b1.reshape(_E, 1, _H),
      W2, b2.reshape(_E, 1, _DOUT),
      w_slot.reshape(_P, 1))


def _sc_combine(go, s_a, s_b):
    mesh = plsc.VectorSubcoreMesh(core_axis_name="c", subcore_axis_name="s")

    @functools.partial(
        pl.kernel,
        mesh=mesh,
        out_type=jax.ShapeDtypeStruct((_T, _DOUT), jnp.float32),
        scratch_types=[
            pltpu.VMEM((_CCH,), jnp.int32),
            pltpu.VMEM((_CCH,), jnp.int32),
            pltpu.VMEM((_CCH, _DOUT), jnp.float32),
            pltpu.VMEM((_CCH, _DOUT), jnp.float32),
            pltpu.SemaphoreType.DMA,
            pltpu.SemaphoreType.DMA,
        ],
    )
    def k(go_hbm, sa_hbm, sb_hbm, out_hbm, ia_v, ib_v, ba_v, bb_v, sema, semb):
        wid = lax.axis_index("s") * 2 + lax.axis_index("c")
        base = wid * _TOK_PER_W

        def chunk(i, carry):
            b = base + i * _CCH
            pltpu.sync_copy(sa_hbm.at[pl.ds(b, _CCH)], ia_v)
            pltpu.sync_copy(sb_hbm.at[pl.ds(b, _CCH)], ib_v)
            cpa = pltpu.async_copy(go_hbm.at[ia_v], ba_v, sema)
            cpb = pltpu.async_copy(go_hbm.at[ib_v], bb_v, semb)
            cpa.wait()
            cpb.wait()

            def row(r, c):
                for j in range(_DOUT // 16):
                    sl = pl.ds(j * 16, 16)
                    ba_v[r, sl] = ba_v[r, sl] + bb_v[r, sl]
                return c

            lax.fori_loop(0, _CCH, row, 0)
            pltpu.sync_copy(ba_v, out_hbm.at[pl.ds(b, _CCH)])
            return carry

        lax.fori_loop(0, _TOK_PER_W // _CCH, chunk, 0)

    return k(go, s_a, s_b)


def kernel(x, Wg, bg, ln_g, ln_b, W1, b1, W2, b2):
    w1, w2, sa, sb, be, lbl = _gate(x, Wg, bg)
    s_a, s_b = sa.reshape(_T), sb.reshape(_T)
    slot_cat = jnp.concatenate([s_a, s_b])
    w_cat = jnp.concatenate([w1.reshape(_T), w2.reshape(_T)])
    w_slot = jnp.zeros((_P,), jnp.float32).at[slot_cat].set(w_cat)
    gx = _sc_dispatch(x, s_a, s_b)
    go = _ffn(gx, ln_g, ln_b, W1, b1, W2, b2, w_slot, be.reshape(_NB))
    final = _sc_combine(go, s_a, s_b)
    return final, lbl[0, 0]
